# baseline (speedup 1.0000x reference)
import functools

import jax
import jax.numpy as jnp
from jax import lax
from jax.experimental import pallas as pl
from jax.experimental.pallas import tpu as pltpu
from jax.experimental.pallas import tpu_sc as plsc

N = 65536
C = 80
TOTAL = N * C
L = 16

R_SC = 8192
R_TC = N - R_SC
NW = 32
CK = R_SC // NW
NV = CK // L

BLK = 8192
G_TC = R_TC // BLK
SUBC = 512

_LOG1P_C = (1.6936626598407223e-06, 0.9998325947816316, -0.49720333122019134,
            0.31504127990864345, -0.18901954822291905, 0.08152317761736225,
            -0.017029610589052675)


def _log1p01(u):
    p = jnp.float32(_LOG1P_C[6])
    for c in _LOG1P_C[5::-1]:
        p = p * u + jnp.float32(c)
    return p


def _pow15_sc(a):
    i = lax.bitcast_convert_type(a, jnp.int32)
    y = lax.bitcast_convert_type(
        jnp.int32(0x5F3759DF) - lax.shift_right_arithmetic(i, 1), jnp.float32)
    y = y * (1.5 - 0.5 * a * y * y)
    y = y * (1.5 - 0.5 * a * y * y)
    return a * a * y


def _loss_tc(x, pos, sc):
    ax = jnp.abs(x)
    u = jnp.exp(-ax)
    d = 1.0 / (1.0 + u)
    sp = jnp.maximum(x, 0.0) + jnp.log1p(u)
    s = jnp.where(x >= 0.0, d, 1.0 - d)
    scm = jnp.where(pos, sc, 0.0)
    a = jnp.where(pos, jnp.abs(sc - s), s)
    return (sp - x * scm) * (a * lax.sqrt(a))


def _loss_sc(x, pos, sc):
    ax = jnp.abs(x)
    u = jnp.exp(-ax)
    d = 1.0 / (1.0 + u)
    sp = jnp.maximum(x, 0.0) + _log1p01(u)
    s = jnp.where(x >= 0.0, d, 1.0 - d)
    scm = jnp.where(pos, sc, 0.0)
    a = jnp.where(pos, jnp.abs(sc - s), s)
    return (sp - x * scm) * _pow15_sc(a)


def _sc_body(pred_h, lab_h, scb_h, out_h, pb, lb, sc_v, acc_v, sem, sem2):
    wid = lax.axis_index("s") * 2 + lax.axis_index("c")
    col0 = wid * CK

    pltpu.sync_copy(scb_h, sc_v)
    cp_p = pltpu.async_copy(pred_h.at[:, pl.ds(col0, CK)], pb, sem)
    cp_l = pltpu.async_copy(lab_h.at[:, pl.ds(col0, CK)], lb, sem2)
    cp_p.wait()
    cp_l.wait()

    def cls(c, acc):
        sc = sc_v[c, :]

        def vec(k, acc):
            for kk in range(4):
                x = pb[c, pl.ds((k * 4 + kk) * L, L)]
                lv = lb[c, pl.ds((k * 4 + kk) * L, L)]
                acc = acc + _loss_sc(x, lv > 0, sc)
            return acc

        return lax.fori_loop(0, NV // 4, vec, acc)

    acc = lax.fori_loop(0, C, cls, jnp.zeros((L,), jnp.float32))

    acc_v[...] = acc
    pltpu.sync_copy(acc_v, out_h.at[pl.ds(wid * L, L)])


def _sc_call(pred_sc, lab_sc, score_b):
    mesh = plsc.VectorSubcoreMesh(core_axis_name="c", subcore_axis_name="s")
    f = functools.partial(
        pl.kernel,
        mesh=mesh,
        out_type=jax.ShapeDtypeStruct((NW * L,), jnp.float32),
        scratch_types=[
            pltpu.VMEM((C, CK), jnp.float32),
            pltpu.VMEM((C, CK), jnp.int32),
            pltpu.VMEM((C, L), jnp.float32),
            pltpu.VMEM((L,), jnp.float32),
            pltpu.SemaphoreType.DMA,
            pltpu.SemaphoreType.DMA,
        ],
    )(_sc_body)
    return f(pred_sc, lab_sc, score_b)


CHC = 4096
NCH_TC = R_TC // CHC


def _tc_chunk(pb, lb, score_ref, acc):
    for j in range(C // 8):
        sc = score_ref[pl.ds(8 * j, 8), :]
        for k in range(CHC // SUBC):
            x = pb[pl.ds(8 * j, 8), pl.ds(k * SUBC, SUBC)]
            pos = lb[pl.ds(8 * j, 8), pl.ds(k * SUBC, SUBC)] > 0
            acc = acc + _loss_tc(x, pos, sc)
    return acc


def _tc_body(score_ref, pred_ref, lab_ref, out_ref,
             pb0, pb1, lb0, lb1, sp0, sp1, sl0, sl1):
    last = (NCH_TC - 1) * CHC

    def start(c_off, pb, lb, sp, sl):
        off = jnp.minimum(c_off, last)
        pltpu.make_async_copy(pred_ref.at[:, pl.ds(off, CHC)], pb, sp).start()
        pltpu.make_async_copy(lab_ref.at[:, pl.ds(off, CHC)], lb, sl).start()

    def wait(pb, lb, sp, sl):
        pltpu.make_async_copy(pred_ref.at[:, pl.ds(0, CHC)], pb, sp).wait()
        pltpu.make_async_copy(lab_ref.at[:, pl.ds(0, CHC)], lb, sl).wait()

    start(0, pb0, lb0, sp0, sl0)
    start(CHC, pb1, lb1, sp1, sl1)

    def pair(g, acc):
        c0 = (2 * g) * CHC
        wait(pb0, lb0, sp0, sl0)
        acc = _tc_chunk(pb0, lb0, score_ref, acc)
        start(c0 + 2 * CHC, pb0, lb0, sp0, sl0)
        wait(pb1, lb1, sp1, sl1)
        acc = _tc_chunk(pb1, lb1, score_ref, acc)
        start(c0 + 3 * CHC, pb1, lb1, sp1, sl1)
        return acc

    acc = lax.fori_loop(0, NCH_TC // 2, pair, jnp.zeros((8, SUBC), jnp.float32))

    wait(pb0, lb0, sp0, sl0)
    wait(pb1, lb1, sp1, sl1)

    out_ref[0, 0] = jnp.sum(acc)


def _tc_call(predT, labT, score):
    return pl.pallas_call(
        _tc_body,
        in_specs=[
            pl.BlockSpec(memory_space=pltpu.VMEM),
            pl.BlockSpec(memory_space=pl.ANY),
            pl.BlockSpec(memory_space=pl.ANY),
        ],
        out_specs=pl.BlockSpec(memory_space=pltpu.SMEM),
        out_shape=jax.ShapeDtypeStruct((1, 1), jnp.float32),
        scratch_shapes=[
            pltpu.VMEM((C, CHC), jnp.float32),
            pltpu.VMEM((C, CHC), jnp.float32),
            pltpu.VMEM((C, CHC), jnp.int32),
            pltpu.VMEM((C, CHC), jnp.int32),
            pltpu.SemaphoreType.DMA,
            pltpu.SemaphoreType.DMA,
            pltpu.SemaphoreType.DMA,
            pltpu.SemaphoreType.DMA,
        ],
    )(score.reshape(C, 1), predT, labT)


@jax.jit
def kernel(pred, label, score):
    predT = pred.T
    labT = label.T
    score_b = jnp.tile(score.reshape(C, 1), (1, L))
    sc_part = _sc_call(predT[:, R_TC:], labT[:, R_TC:], score_b)
    tc_part = _tc_call(predT, labT, score)
    return (jnp.sum(tc_part) + jnp.sum(sc_part)) / jnp.float32(TOTAL)


# speedup vs baseline: 1.0452x; 1.0452x over previous
import functools

import jax
import jax.numpy as jnp
from jax import lax
from jax.experimental import pallas as pl
from jax.experimental.pallas import tpu as pltpu
from jax.experimental.pallas import tpu_sc as plsc

N = 65536
C = 80
TOTAL = N * C
L = 16

R_SC = 8192
R_TC = N - R_SC
NW = 32
CK = R_SC // NW
NV = CK // L

BLK = 8192
G_TC = R_TC // BLK
SUBC = 512

_LOG1P_C = (1.6936626598407223e-06, 0.9998325947816316, -0.49720333122019134,
            0.31504127990864345, -0.18901954822291905, 0.08152317761736225,
            -0.017029610589052675)


def _log1p01(u):
    p = jnp.float32(_LOG1P_C[6])
    for c in _LOG1P_C[5::-1]:
        p = p * u + jnp.float32(c)
    return p


def _pow15_sc(a):
    i = lax.bitcast_convert_type(a, jnp.int32)
    y = lax.bitcast_convert_type(
        jnp.int32(0x5F3759DF) - lax.shift_right_arithmetic(i, 1), jnp.float32)
    y = y * (1.5 - 0.5 * a * y * y)
    y = y * (1.5 - 0.5 * a * y * y)
    return a * a * y


def _loss_tc(x, pos, sc):
    ax = jnp.abs(x)
    u = jnp.exp(-ax)
    d = 1.0 / (1.0 + u)
    sp = jnp.maximum(x, 0.0) + jnp.log1p(u)
    s = jnp.where(x >= 0.0, d, 1.0 - d)
    scm = jnp.where(pos, sc, 0.0)
    a = jnp.where(pos, jnp.abs(sc - s), s)
    return (sp - x * scm) * (a * lax.sqrt(a))


def _loss_sc(x, pos, sc):
    ax = jnp.abs(x)
    u = jnp.exp(-ax)
    d = 1.0 / (1.0 + u)
    sp = jnp.maximum(x, 0.0) + _log1p01(u)
    s = jnp.where(x >= 0.0, d, 1.0 - d)
    scm = jnp.where(pos, sc, 0.0)
    a = jnp.where(pos, jnp.abs(sc - s), s)
    return (sp - x * scm) * _pow15_sc(a)


def _sc_body(pred_h, lab_h, scb_h, out_h, pb, lb, sc_v, acc_v, sem, sem2):
    wid = lax.axis_index("s") * 2 + lax.axis_index("c")
    col0 = wid * CK

    pltpu.sync_copy(scb_h, sc_v)
    cp_p = pltpu.async_copy(pred_h.at[:, pl.ds(col0, CK)], pb, sem)
    cp_l = pltpu.async_copy(lab_h.at[:, pl.ds(col0, CK)], lb, sem2)
    cp_p.wait()
    cp_l.wait()

    def cls(c, acc):
        sc = sc_v[c, :]

        def vec(k, acc):
            for kk in range(4):
                x = pb[c, pl.ds((k * 4 + kk) * L, L)]
                lv = lb[c, pl.ds((k * 4 + kk) * L, L)]
                acc = acc + _loss_sc(x, lv > 0, sc)
            return acc

        return lax.fori_loop(0, NV // 4, vec, acc)

    acc = lax.fori_loop(0, C, cls, jnp.zeros((L,), jnp.float32))

    acc_v[...] = acc
    pltpu.sync_copy(acc_v, out_h.at[pl.ds(wid * L, L)])


def _sc_call(pred_sc, lab_sc, score_b):
    mesh = plsc.VectorSubcoreMesh(core_axis_name="c", subcore_axis_name="s")
    f = functools.partial(
        pl.kernel,
        mesh=mesh,
        out_type=jax.ShapeDtypeStruct((NW * L,), jnp.float32),
        scratch_types=[
            pltpu.VMEM((C, CK), jnp.float32),
            pltpu.VMEM((C, CK), jnp.int32),
            pltpu.VMEM((C, L), jnp.float32),
            pltpu.VMEM((L,), jnp.float32),
            pltpu.SemaphoreType.DMA,
            pltpu.SemaphoreType.DMA,
        ],
    )(_sc_body)
    return f(pred_sc, lab_sc, score_b)


def _tc_body(score_ref, pred_ref, lab_ref, out_ref):
    sc = score_ref[...]
    acc = jnp.zeros((8, SUBC), jnp.float32)
    for k in range(R_TC // SUBC):
        x = pred_ref[:, pl.ds(k * SUBC, SUBC)]
        pos = lab_ref[:, pl.ds(k * SUBC, SUBC)] > 0
        acc = acc + _loss_tc(x, pos, sc)

    @pl.when(pl.program_id(0) == 0)
    def _():
        out_ref[0, 0] = 0.0

    out_ref[0, 0] += jnp.sum(acc)


def _tc_call(predT, labT, score):
    return pl.pallas_call(
        _tc_body,
        grid=(C // 8,),
        in_specs=[
            pl.BlockSpec((8, 1), lambda i: (i, 0)),
            pl.BlockSpec((8, R_TC), lambda i: (i, 0)),
            pl.BlockSpec((8, R_TC), lambda i: (i, 0)),
        ],
        out_specs=pl.BlockSpec((1, 1), lambda i: (0, 0),
                               memory_space=pltpu.SMEM),
        out_shape=jax.ShapeDtypeStruct((1, 1), jnp.float32),
    )(score.reshape(C, 1), predT, labT)


@jax.jit
def kernel(pred, label, score):
    predT = pred.T
    labT = label.T
    score_b = jnp.tile(score.reshape(C, 1), (1, L))
    sc_part = _sc_call(predT[:, R_TC:], labT[:, R_TC:], score_b)
    tc_part = _tc_call(predT, labT, score)
    return (jnp.sum(tc_part) + jnp.sum(sc_part)) / jnp.float32(TOTAL)
